# baseline (speedup 1.0000x reference)
import jax
import jax.numpy as jnp
from jax.experimental import pallas as pl
from jax.experimental.pallas import tpu as pltpu

N = 100000
D_IN = 128
D_HID = 256
D_OUT = 128
BLOCK = 5000
K_IN = 6
K_OUT = 4
STEPS = N // BLOCK


def _outer(x_hbm, w1_ref, b1_ref, w2_ref, b2_ref, out_hbm, ibuf, obuf, isem, osem):
    w1b = w1_ref[...].astype(jnp.bfloat16)
    w2b = w2_ref[...].astype(jnp.bfloat16)
    b1v = b1_ref[...]
    b2v = b2_ref[...]

    def _in_copy(step):
        slot = jax.lax.rem(step, K_IN)
        return pltpu.make_async_copy(
            x_hbm.at[pl.ds(step * BLOCK, BLOCK), :],
            ibuf.at[slot],
            isem.at[slot],
        )

    def _out_copy(step):
        slot = jax.lax.rem(step, K_OUT)
        return pltpu.make_async_copy(
            obuf.at[slot],
            out_hbm.at[pl.ds(step * BLOCK, BLOCK), :],
            osem.at[slot],
        )

    for j in range(K_IN):
        _in_copy(j).start()

    def body(i, _):
        islot = jax.lax.rem(i, K_IN)
        oslot = jax.lax.rem(i, K_OUT)

        @pl.when(i >= K_OUT)
        def _wait_out():
            _out_copy(i - K_OUT).wait()

        _in_copy(i).wait()
        xb = ibuf[islot].astype(jnp.bfloat16)
        h = jnp.dot(xb, w1b, preferred_element_type=jnp.float32)
        h = jnp.maximum(h + b1v, 0.0).astype(jnp.bfloat16)
        out = jnp.dot(h, w2b, preferred_element_type=jnp.float32)
        obuf[oslot] = out + b2v
        _out_copy(i).start()

        @pl.when(i + K_IN < STEPS)
        def _prefetch():
            _in_copy(i + K_IN).start()

        return 0

    jax.lax.fori_loop(0, STEPS, body, 0)

    for j in range(max(0, STEPS - K_OUT), STEPS):
        _out_copy(j).wait()


@jax.jit
def kernel(x, W1, b1, W2, b2):
    b1r = b1.reshape(1, D_HID)
    b2r = b2.reshape(1, D_OUT)
    return pl.pallas_call(
        _outer,
        in_specs=[
            pl.BlockSpec(memory_space=pltpu.MemorySpace.HBM),
            pl.BlockSpec(memory_space=pltpu.MemorySpace.VMEM),
            pl.BlockSpec(memory_space=pltpu.MemorySpace.VMEM),
            pl.BlockSpec(memory_space=pltpu.MemorySpace.VMEM),
            pl.BlockSpec(memory_space=pltpu.MemorySpace.VMEM),
        ],
        out_specs=pl.BlockSpec(memory_space=pltpu.MemorySpace.HBM),
        out_shape=jax.ShapeDtypeStruct((N, D_OUT), jnp.float32),
        scratch_shapes=[
            pltpu.VMEM((K_IN, BLOCK, D_IN), jnp.float32),
            pltpu.VMEM((K_OUT, BLOCK, D_OUT), jnp.float32),
            pltpu.SemaphoreType.DMA((K_IN,)),
            pltpu.SemaphoreType.DMA((K_OUT,)),
        ],
    )(x, W1, b1r, W2, b2r)


# speedup vs baseline: 1.0418x; 1.0418x over previous
import jax
import jax.numpy as jnp
from jax.experimental import pallas as pl
from jax.experimental.pallas import tpu as pltpu

N = 100000
D_IN = 128
D_HID = 256
D_OUT = 128
BLOCK = 10000
K_IN = 4
K_OUT = 3
STEPS = N // BLOCK


def _outer(x_hbm, w1_ref, b1_ref, w2_ref, b2_ref, out_hbm, ibuf, obuf, isem, osem):
    w1b = w1_ref[...].astype(jnp.bfloat16)
    w2b = w2_ref[...].astype(jnp.bfloat16)
    b1v = b1_ref[...]
    b2v = b2_ref[...]

    def _in_copy(step):
        slot = jax.lax.rem(step, K_IN)
        return pltpu.make_async_copy(
            x_hbm.at[pl.ds(step * BLOCK, BLOCK), :],
            ibuf.at[slot],
            isem.at[slot],
        )

    def _out_copy(step):
        slot = jax.lax.rem(step, K_OUT)
        return pltpu.make_async_copy(
            obuf.at[slot],
            out_hbm.at[pl.ds(step * BLOCK, BLOCK), :],
            osem.at[slot],
        )

    for j in range(K_IN):
        _in_copy(j).start()

    def body(i, _):
        islot = jax.lax.rem(i, K_IN)
        oslot = jax.lax.rem(i, K_OUT)

        @pl.when(i >= K_OUT)
        def _wait_out():
            _out_copy(i - K_OUT).wait()

        _in_copy(i).wait()
        xb = ibuf[islot].astype(jnp.bfloat16)
        h = jnp.dot(xb, w1b, preferred_element_type=jnp.float32)
        h = jnp.maximum(h + b1v, 0.0).astype(jnp.bfloat16)
        out = jnp.dot(h, w2b, preferred_element_type=jnp.float32)
        obuf[oslot] = out + b2v
        _out_copy(i).start()

        @pl.when(i + K_IN < STEPS)
        def _prefetch():
            _in_copy(i + K_IN).start()

        return 0

    jax.lax.fori_loop(0, STEPS, body, 0)

    for j in range(max(0, STEPS - K_OUT), STEPS):
        _out_copy(j).wait()


@jax.jit
def kernel(x, W1, b1, W2, b2):
    b1r = b1.reshape(1, D_HID)
    b2r = b2.reshape(1, D_OUT)
    return pl.pallas_call(
        _outer,
        in_specs=[
            pl.BlockSpec(memory_space=pltpu.MemorySpace.HBM),
            pl.BlockSpec(memory_space=pltpu.MemorySpace.VMEM),
            pl.BlockSpec(memory_space=pltpu.MemorySpace.VMEM),
            pl.BlockSpec(memory_space=pltpu.MemorySpace.VMEM),
            pl.BlockSpec(memory_space=pltpu.MemorySpace.VMEM),
        ],
        out_specs=pl.BlockSpec(memory_space=pltpu.MemorySpace.HBM),
        out_shape=jax.ShapeDtypeStruct((N, D_OUT), jnp.float32),
        scratch_shapes=[
            pltpu.VMEM((K_IN, BLOCK, D_IN), jnp.float32),
            pltpu.VMEM((K_OUT, BLOCK, D_OUT), jnp.float32),
            pltpu.SemaphoreType.DMA((K_IN,)),
            pltpu.SemaphoreType.DMA((K_OUT,)),
        ],
    )(x, W1, b1r, W2, b2r)
